# baseline (speedup 1.0000x reference)
import functools

import jax
import jax.numpy as jnp
from jax import lax
from jax.experimental import pallas as pl
from jax.experimental.pallas import tpu as pltpu
from jax.experimental.pallas import tpu_sc as plsc

V = 10000
D = 128
T = 4
E = 80000

NC = 2
NS = 16
CHUNK = 40
TYPES_PER_SC = T // NC
EPT = E // NS
NJ = EPT // CHUNK
NBUF = 6
STAGE_OFF = 4992
STAGE_LEN = 5120
ZROWS = 48

ROWS_MAIN = 624
ROWS_TAIL = V - ROWS_MAIN * NS


def _sc_message_passing(node_values, edges_r, zeros):
    mesh = plsc.VectorSubcoreMesh(core_axis_name="c", subcore_axis_name="s")

    @functools.partial(
        pl.kernel,
        out_type=jax.ShapeDtypeStruct((V, T * D), jnp.float32),
        mesh=mesh,
        scratch_types=[
            pltpu.VMEM_SHARED((V, D), jnp.float32),
            pltpu.VMEM((STAGE_LEN,), jnp.int32),
            pltpu.VMEM((STAGE_LEN,), jnp.int32),
            pltpu.VMEM((NBUF, CHUNK, D), jnp.float32),
            pltpu.VMEM((ZROWS, D), jnp.float32),
            pltpu.SemaphoreType.DMA((NBUF,)),
            pltpu.SemaphoreType.DMA((NBUF,)),
            pltpu.SemaphoreType.DMA,
        ],
    )
    def sc_kernel(node_hbm, edges_hbm, zeros_hbm, msgs_hbm, acc, sidx, didx,
                  rows, zbuf, gsem, ssem, zsem):
        c = lax.axis_index("c")
        s = lax.axis_index("s")
        base = s * ROWS_MAIN
        ioff = 8 * s

        pltpu.sync_copy(zeros_hbm, zbuf)

        def gather(j, r):
            pltpu.async_copy(
                node_hbm.at[sidx.at[pl.ds(ioff + j * CHUNK, CHUNK)]],
                rows.at[r], gsem.at[r])

        def gather_wait(j, r):
            pltpu.make_async_copy(
                node_hbm.at[sidx.at[pl.ds(ioff + j * CHUNK, CHUNK)]],
                rows.at[r], gsem.at[r]).wait()

        def scatter(j, r):
            pltpu.async_copy(rows.at[r],
                             acc.at[didx.at[pl.ds(ioff + j * CHUNK, CHUNK)]],
                             ssem.at[r], add=True)

        def scatter_wait(j, r):
            pltpu.make_async_copy(
                rows.at[r], acc.at[didx.at[pl.ds(ioff + j * CHUNK, CHUNK)]],
                ssem.at[r]).wait()

        for tt in range(TYPES_PER_SC):
            t = c * TYPES_PER_SC + tt

            for k in range(ROWS_MAIN // ZROWS):
                pltpu.async_copy(zbuf, acc.at[pl.ds(base + k * ZROWS, ZROWS)],
                                 zsem)

            @pl.when(s == NS - 1)
            def _():
                pltpu.async_copy(zbuf.at[pl.ds(0, ROWS_TAIL)],
                                 acc.at[pl.ds(ROWS_MAIN * NS, ROWS_TAIL)],
                                 zsem)

            pltpu.async_copy(
                edges_hbm.at[t, 0, pl.ds(STAGE_OFF * s, STAGE_LEN)],
                sidx, gsem.at[0])
            pltpu.async_copy(
                edges_hbm.at[t, 1, pl.ds(STAGE_OFF * s, STAGE_LEN)],
                didx, gsem.at[1])
            pltpu.make_async_copy(
                edges_hbm.at[t, 0, pl.ds(STAGE_OFF * s, STAGE_LEN)],
                sidx, gsem.at[0]).wait()
            pltpu.make_async_copy(
                edges_hbm.at[t, 1, pl.ds(STAGE_OFF * s, STAGE_LEN)],
                didx, gsem.at[1]).wait()

            for k in range(ROWS_MAIN // ZROWS):
                pltpu.make_async_copy(
                    zbuf, acc.at[pl.ds(base + k * ZROWS, ZROWS)], zsem).wait()

            @pl.when(s == NS - 1)
            def _():
                pltpu.make_async_copy(
                    zbuf.at[pl.ds(0, ROWS_TAIL)],
                    acc.at[pl.ds(ROWS_MAIN * NS, ROWS_TAIL)], zsem).wait()

            plsc.subcore_barrier()

            for r in range(NBUF):
                gather(r, r)

            @pl.loop(0, pl.cdiv(NJ, NBUF))
            def _(kk):
                for r in range(NBUF):
                    j = kk * NBUF + r

                    @pl.when(j < NJ)
                    def _():
                        gather_wait(j, r)
                        scatter(j, r)

                    @pl.when(j + NBUF < NJ)
                    def _():
                        scatter_wait(j, r)
                        gather(j + NBUF, r)

            for r in range(NBUF):
                scatter_wait(0, r)

            plsc.subcore_barrier()

            pltpu.sync_copy(acc.at[pl.ds(base, ROWS_MAIN)],
                            msgs_hbm.at[pl.ds(base, ROWS_MAIN),
                                        pl.ds(t * D, D)])

            @pl.when(s == NS - 1)
            def _():
                pltpu.sync_copy(
                    acc.at[pl.ds(ROWS_MAIN * NS, ROWS_TAIL)],
                    msgs_hbm.at[pl.ds(ROWS_MAIN * NS, ROWS_TAIL),
                                pl.ds(t * D, D)])


    return sc_kernel(node_values, edges_r, zeros)


BV = 1000


def _mm_body(msgs_ref, w_ref, b_ref, out_ref):
    out_ref[...] = b_ref[...] + jnp.dot(msgs_ref[...], w_ref[...],
                                        preferred_element_type=jnp.float32)


def _tc_matmul(msgs, W, b2):
    return pl.pallas_call(
        _mm_body,
        grid=(V // BV,),
        in_specs=[
            pl.BlockSpec((BV, T * D), lambda i: (i, 0)),
            pl.BlockSpec((T * D, D), lambda i: (0, 0)),
            pl.BlockSpec((1, D), lambda i: (0, 0)),
        ],
        out_specs=pl.BlockSpec((BV, D), lambda i: (i, 0)),
        out_shape=jax.ShapeDtypeStruct((V, D), jnp.float32),
        compiler_params=pltpu.CompilerParams(
            dimension_semantics=("parallel",)),
    )(msgs, W, b2)


def kernel(node_values, edges, W, b):
    edges_r = edges.astype(jnp.int32)
    zeros = jnp.zeros((ZROWS, D), jnp.float32)
    msgs = _sc_message_passing(node_values, edges_r, zeros)
    return _tc_matmul(msgs, W, b.reshape(1, D))


# speedup vs baseline: 1.0265x; 1.0265x over previous
import functools

import jax
import jax.numpy as jnp
from jax import lax
from jax.experimental import pallas as pl
from jax.experimental.pallas import tpu as pltpu
from jax.experimental.pallas import tpu_sc as plsc

V = 10000
D = 128
T = 4
E = 80000

NC = 2
NS = 16
CHUNK = 40
TYPES_PER_SC = T // NC
EPT = E // NS
NJ = EPT // CHUNK
NBUF = 6
STAGE_OFF = 4992
STAGE_LEN = 5120
ZROWS = 48

ROWS_MAIN = 624
ROWS_TAIL = V - ROWS_MAIN * NS


def _sc_message_passing(node_values, edges_r, zeros):
    mesh = plsc.VectorSubcoreMesh(core_axis_name="c", subcore_axis_name="s")

    @functools.partial(
        pl.kernel,
        out_type=jax.ShapeDtypeStruct((V, T * D), jnp.float32),
        mesh=mesh,
        scratch_types=[
            pltpu.VMEM_SHARED((V, D), jnp.float32),
            pltpu.VMEM((STAGE_LEN,), jnp.int32),
            pltpu.VMEM((STAGE_LEN,), jnp.int32),
            pltpu.VMEM((NBUF, CHUNK, D), jnp.float32),
            pltpu.VMEM((ZROWS, D), jnp.float32),
            pltpu.SemaphoreType.DMA((NBUF,)),
            pltpu.SemaphoreType.DMA((NBUF,)),
            pltpu.SemaphoreType.DMA,
        ],
    )
    def sc_kernel(node_hbm, edges_hbm, zeros_hbm, msgs_hbm, acc, sidx, didx,
                  rows, zbuf, gsem, ssem, zsem):
        c = lax.axis_index("c")
        s = lax.axis_index("s")
        base = s * ROWS_MAIN
        ioff = 8 * s

        pltpu.sync_copy(zeros_hbm, zbuf)

        def gather(j, r):
            pltpu.async_copy(
                node_hbm.at[sidx.at[pl.ds(ioff + j * CHUNK, CHUNK)]],
                rows.at[r], gsem.at[r])

        def gather_wait(j, r):
            pltpu.make_async_copy(
                node_hbm.at[sidx.at[pl.ds(ioff + j * CHUNK, CHUNK)]],
                rows.at[r], gsem.at[r]).wait()

        def scatter(j, r):
            pltpu.async_copy(rows.at[r],
                             acc.at[didx.at[pl.ds(ioff + j * CHUNK, CHUNK)]],
                             ssem.at[r], add=True)

        def scatter_wait(j, r):
            pltpu.make_async_copy(
                rows.at[r], acc.at[didx.at[pl.ds(ioff + j * CHUNK, CHUNK)]],
                ssem.at[r]).wait()

        for tt in range(TYPES_PER_SC):
            t = c * TYPES_PER_SC + tt

            for k in range(ROWS_MAIN // ZROWS):
                pltpu.async_copy(zbuf, acc.at[pl.ds(base + k * ZROWS, ZROWS)],
                                 zsem)

            @pl.when(s == NS - 1)
            def _():
                pltpu.async_copy(zbuf.at[pl.ds(0, ROWS_TAIL)],
                                 acc.at[pl.ds(ROWS_MAIN * NS, ROWS_TAIL)],
                                 zsem)

            pltpu.async_copy(
                edges_hbm.at[t, 0, pl.ds(STAGE_OFF * s, STAGE_LEN)],
                sidx, gsem.at[0])
            pltpu.async_copy(
                edges_hbm.at[t, 1, pl.ds(STAGE_OFF * s, STAGE_LEN)],
                didx, gsem.at[1])
            pltpu.make_async_copy(
                edges_hbm.at[t, 0, pl.ds(STAGE_OFF * s, STAGE_LEN)],
                sidx, gsem.at[0]).wait()
            pltpu.make_async_copy(
                edges_hbm.at[t, 1, pl.ds(STAGE_OFF * s, STAGE_LEN)],
                didx, gsem.at[1]).wait()

            for k in range(ROWS_MAIN // ZROWS):
                pltpu.make_async_copy(
                    zbuf, acc.at[pl.ds(base + k * ZROWS, ZROWS)], zsem).wait()

            @pl.when(s == NS - 1)
            def _():
                pltpu.make_async_copy(
                    zbuf.at[pl.ds(0, ROWS_TAIL)],
                    acc.at[pl.ds(ROWS_MAIN * NS, ROWS_TAIL)], zsem).wait()

            plsc.subcore_barrier()

            for r in range(NBUF):
                gather(r, r)

            @pl.loop(0, pl.cdiv(NJ, NBUF))
            def _(kk):
                for r in range(NBUF):
                    j = kk * NBUF + r

                    @pl.when(j < NJ)
                    def _():
                        gather_wait(j, r)
                        scatter(j, r)

                    @pl.when(j + NBUF < NJ)
                    def _():
                        scatter_wait(j, r)
                        gather(j + NBUF, r)

            for r in range(NBUF):
                scatter_wait(0, r)

            plsc.subcore_barrier()

            pltpu.sync_copy(acc.at[pl.ds(base, ROWS_MAIN)],
                            msgs_hbm.at[pl.ds(base, ROWS_MAIN),
                                        pl.ds(t * D, D)])

            @pl.when(s == NS - 1)
            def _():
                pltpu.sync_copy(
                    acc.at[pl.ds(ROWS_MAIN * NS, ROWS_TAIL)],
                    msgs_hbm.at[pl.ds(ROWS_MAIN * NS, ROWS_TAIL),
                                pl.ds(t * D, D)])


    return sc_kernel(node_values, edges_r, zeros)


BV = 5000


def _mm_body(msgs_ref, w_ref, b_ref, out_ref):
    out_ref[...] = b_ref[...] + jnp.dot(msgs_ref[...], w_ref[...],
                                        preferred_element_type=jnp.float32)


def _tc_matmul(msgs, W, b2):
    return pl.pallas_call(
        _mm_body,
        grid=(V // BV,),
        in_specs=[
            pl.BlockSpec((BV, T * D), lambda i: (i, 0)),
            pl.BlockSpec((T * D, D), lambda i: (0, 0)),
            pl.BlockSpec((1, D), lambda i: (0, 0)),
        ],
        out_specs=pl.BlockSpec((BV, D), lambda i: (i, 0)),
        out_shape=jax.ShapeDtypeStruct((V, D), jnp.float32),
        compiler_params=pltpu.CompilerParams(
            dimension_semantics=("parallel",)),
    )(msgs, W, b2)


def kernel(node_values, edges, W, b):
    edges_r = edges.astype(jnp.int32)
    zeros = jnp.zeros((ZROWS, D), jnp.float32)
    msgs = _sc_message_passing(node_values, edges_r, zeros)
    return _tc_matmul(msgs, W, b.reshape(1, D))


# speedup vs baseline: 1.0338x; 1.0071x over previous
import functools

import jax
import jax.numpy as jnp
from jax import lax
from jax.experimental import pallas as pl
from jax.experimental.pallas import tpu as pltpu
from jax.experimental.pallas import tpu_sc as plsc

V = 10000
D = 128
T = 4
E = 80000

NC = 2
NS = 16
CHUNK = 40
TYPES_PER_SC = T // NC
EPT = E // NS
NJ = EPT // CHUNK
NBUF = 6
STAGE_OFF = 4992
STAGE_LEN = 5120
ZROWS = 48

ROWS_MAIN = 624
ROWS_TAIL = V - ROWS_MAIN * NS


def _sc_message_passing(node_values, edges_r, zeros):
    mesh = plsc.VectorSubcoreMesh(core_axis_name="c", subcore_axis_name="s")

    @functools.partial(
        pl.kernel,
        out_type=jax.ShapeDtypeStruct((V, T * D), jnp.float32),
        mesh=mesh,
        scratch_types=[
            pltpu.VMEM_SHARED((V, D), jnp.float32),
            pltpu.VMEM((STAGE_LEN,), jnp.int32),
            pltpu.VMEM((STAGE_LEN,), jnp.int32),
            pltpu.VMEM((NBUF, CHUNK, D), jnp.float32),
            pltpu.VMEM((ZROWS, D), jnp.float32),
            pltpu.SemaphoreType.DMA((NBUF,)),
            pltpu.SemaphoreType.DMA((NBUF,)),
            pltpu.SemaphoreType.DMA,
        ],
    )
    def sc_kernel(node_hbm, edges_hbm, zeros_hbm, msgs_hbm, acc, sidx, didx,
                  rows, zbuf, gsem, ssem, zsem):
        c = lax.axis_index("c")
        s = lax.axis_index("s")
        base = s * ROWS_MAIN
        ioff = 8 * s

        pltpu.sync_copy(zeros_hbm, zbuf)

        def gather(j, r):
            pltpu.async_copy(
                node_hbm.at[sidx.at[pl.ds(ioff + j * CHUNK, CHUNK)]],
                rows.at[r], gsem.at[r])

        def gather_wait(j, r):
            pltpu.make_async_copy(
                node_hbm.at[sidx.at[pl.ds(ioff + j * CHUNK, CHUNK)]],
                rows.at[r], gsem.at[r]).wait()

        def scatter(j, r):
            pltpu.async_copy(rows.at[r],
                             acc.at[didx.at[pl.ds(ioff + j * CHUNK, CHUNK)]],
                             ssem.at[r], add=True)

        def scatter_wait(j, r):
            pltpu.make_async_copy(
                rows.at[r], acc.at[didx.at[pl.ds(ioff + j * CHUNK, CHUNK)]],
                ssem.at[r]).wait()

        def stage_idx(t):
            pltpu.async_copy(
                edges_hbm.at[t, 0, pl.ds(STAGE_OFF * s, STAGE_LEN)],
                sidx, gsem.at[0])
            pltpu.async_copy(
                edges_hbm.at[t, 1, pl.ds(STAGE_OFF * s, STAGE_LEN)],
                didx, gsem.at[1])

        def stage_idx_wait(t):
            pltpu.make_async_copy(
                edges_hbm.at[t, 0, pl.ds(STAGE_OFF * s, STAGE_LEN)],
                sidx, gsem.at[0]).wait()
            pltpu.make_async_copy(
                edges_hbm.at[t, 1, pl.ds(STAGE_OFF * s, STAGE_LEN)],
                didx, gsem.at[1]).wait()

        stage_idx(c * TYPES_PER_SC)

        for tt in range(TYPES_PER_SC):
            t = c * TYPES_PER_SC + tt

            for k in range(ROWS_MAIN // ZROWS):
                pltpu.async_copy(zbuf, acc.at[pl.ds(base + k * ZROWS, ZROWS)],
                                 zsem)

            @pl.when(s == NS - 1)
            def _():
                pltpu.async_copy(zbuf.at[pl.ds(0, ROWS_TAIL)],
                                 acc.at[pl.ds(ROWS_MAIN * NS, ROWS_TAIL)],
                                 zsem)

            stage_idx_wait(t)

            for k in range(ROWS_MAIN // ZROWS):
                pltpu.make_async_copy(
                    zbuf, acc.at[pl.ds(base + k * ZROWS, ZROWS)], zsem).wait()

            @pl.when(s == NS - 1)
            def _():
                pltpu.make_async_copy(
                    zbuf.at[pl.ds(0, ROWS_TAIL)],
                    acc.at[pl.ds(ROWS_MAIN * NS, ROWS_TAIL)], zsem).wait()

            plsc.subcore_barrier()

            for r in range(NBUF):
                gather(r, r)

            @pl.loop(0, pl.cdiv(NJ, NBUF))
            def _(kk):
                for r in range(NBUF):
                    j = kk * NBUF + r

                    @pl.when(j < NJ)
                    def _():
                        gather_wait(j, r)
                        scatter(j, r)

                    @pl.when(j + NBUF < NJ)
                    def _():
                        scatter_wait(j, r)
                        gather(j + NBUF, r)

            for r in range(NBUF):
                scatter_wait(0, r)

            plsc.subcore_barrier()

            pltpu.async_copy(acc.at[pl.ds(base, ROWS_MAIN)],
                             msgs_hbm.at[pl.ds(base, ROWS_MAIN),
                                         pl.ds(t * D, D)], zsem)

            @pl.when(s == NS - 1)
            def _():
                pltpu.async_copy(
                    acc.at[pl.ds(ROWS_MAIN * NS, ROWS_TAIL)],
                    msgs_hbm.at[pl.ds(ROWS_MAIN * NS, ROWS_TAIL),
                                pl.ds(t * D, D)], zsem)

            if tt + 1 < TYPES_PER_SC:
                stage_idx(t + 1)

            pltpu.make_async_copy(acc.at[pl.ds(base, ROWS_MAIN)],
                                  msgs_hbm.at[pl.ds(base, ROWS_MAIN),
                                              pl.ds(t * D, D)], zsem).wait()

            @pl.when(s == NS - 1)
            def _():
                pltpu.make_async_copy(
                    acc.at[pl.ds(ROWS_MAIN * NS, ROWS_TAIL)],
                    msgs_hbm.at[pl.ds(ROWS_MAIN * NS, ROWS_TAIL),
                                pl.ds(t * D, D)], zsem).wait()


    return sc_kernel(node_values, edges_r, zeros)


BV = 5000


def _mm_body(msgs_ref, w_ref, b_ref, out_ref):
    out_ref[...] = b_ref[...] + jnp.dot(msgs_ref[...], w_ref[...],
                                        preferred_element_type=jnp.float32)


def _tc_matmul(msgs, W, b2):
    return pl.pallas_call(
        _mm_body,
        grid=(V // BV,),
        in_specs=[
            pl.BlockSpec((BV, T * D), lambda i: (i, 0)),
            pl.BlockSpec((T * D, D), lambda i: (0, 0)),
            pl.BlockSpec((1, D), lambda i: (0, 0)),
        ],
        out_specs=pl.BlockSpec((BV, D), lambda i: (i, 0)),
        out_shape=jax.ShapeDtypeStruct((V, D), jnp.float32),
        compiler_params=pltpu.CompilerParams(
            dimension_semantics=("parallel",)),
    )(msgs, W, b2)


def kernel(node_values, edges, W, b):
    edges_r = edges.astype(jnp.int32)
    zeros = jnp.zeros((ZROWS, D), jnp.float32)
    msgs = _sc_message_passing(node_values, edges_r, zeros)
    return _tc_matmul(msgs, W, b.reshape(1, D))


# speedup vs baseline: 1.0366x; 1.0027x over previous
import functools

import jax
import jax.numpy as jnp
from jax import lax
from jax.experimental import pallas as pl
from jax.experimental.pallas import tpu as pltpu
from jax.experimental.pallas import tpu_sc as plsc

V = 10000
D = 128
T = 4
E = 80000

NC = 2
NS = 16
CHUNK = 40
TYPES_PER_SC = T // NC
EPT = E // NS
NJ = EPT // CHUNK
NBUF = 7
STAGE_OFF = 4992
STAGE_LEN = 5120
ZROWS = 16

ROWS_MAIN = 624
ROWS_TAIL = V - ROWS_MAIN * NS


def _sc_message_passing(node_values, edges_r, zeros):
    mesh = plsc.VectorSubcoreMesh(core_axis_name="c", subcore_axis_name="s")

    @functools.partial(
        pl.kernel,
        out_type=jax.ShapeDtypeStruct((V, T * D), jnp.float32),
        mesh=mesh,
        scratch_types=[
            pltpu.VMEM_SHARED((V, D), jnp.float32),
            pltpu.VMEM((STAGE_LEN,), jnp.int32),
            pltpu.VMEM((STAGE_LEN,), jnp.int32),
            pltpu.VMEM((NBUF, CHUNK, D), jnp.float32),
            pltpu.VMEM((ZROWS, D), jnp.float32),
            pltpu.SemaphoreType.DMA((NBUF,)),
            pltpu.SemaphoreType.DMA((NBUF,)),
            pltpu.SemaphoreType.DMA,
        ],
    )
    def sc_kernel(node_hbm, edges_hbm, zeros_hbm, msgs_hbm, acc, sidx, didx,
                  rows, zbuf, gsem, ssem, zsem):
        c = lax.axis_index("c")
        s = lax.axis_index("s")
        base = s * ROWS_MAIN
        ioff = 8 * s

        pltpu.sync_copy(zeros_hbm, zbuf)

        def gather(j, r):
            pltpu.async_copy(
                node_hbm.at[sidx.at[pl.ds(ioff + j * CHUNK, CHUNK)]],
                rows.at[r], gsem.at[r])

        def gather_wait(j, r):
            pltpu.make_async_copy(
                node_hbm.at[sidx.at[pl.ds(ioff + j * CHUNK, CHUNK)]],
                rows.at[r], gsem.at[r]).wait()

        def scatter(j, r):
            pltpu.async_copy(rows.at[r],
                             acc.at[didx.at[pl.ds(ioff + j * CHUNK, CHUNK)]],
                             ssem.at[r], add=True)

        def scatter_wait(j, r):
            pltpu.make_async_copy(
                rows.at[r], acc.at[didx.at[pl.ds(ioff + j * CHUNK, CHUNK)]],
                ssem.at[r]).wait()

        def stage_idx(t):
            pltpu.async_copy(
                edges_hbm.at[t, 0, pl.ds(STAGE_OFF * s, STAGE_LEN)],
                sidx, gsem.at[0])
            pltpu.async_copy(
                edges_hbm.at[t, 1, pl.ds(STAGE_OFF * s, STAGE_LEN)],
                didx, gsem.at[1])

        def stage_idx_wait(t):
            pltpu.make_async_copy(
                edges_hbm.at[t, 0, pl.ds(STAGE_OFF * s, STAGE_LEN)],
                sidx, gsem.at[0]).wait()
            pltpu.make_async_copy(
                edges_hbm.at[t, 1, pl.ds(STAGE_OFF * s, STAGE_LEN)],
                didx, gsem.at[1]).wait()

        stage_idx(c * TYPES_PER_SC)

        for tt in range(TYPES_PER_SC):
            t = c * TYPES_PER_SC + tt

            for k in range(ROWS_MAIN // ZROWS):
                pltpu.async_copy(zbuf, acc.at[pl.ds(base + k * ZROWS, ZROWS)],
                                 zsem)

            @pl.when(s == NS - 1)
            def _():
                pltpu.async_copy(zbuf.at[pl.ds(0, ROWS_TAIL)],
                                 acc.at[pl.ds(ROWS_MAIN * NS, ROWS_TAIL)],
                                 zsem)

            stage_idx_wait(t)

            for k in range(ROWS_MAIN // ZROWS):
                pltpu.make_async_copy(
                    zbuf, acc.at[pl.ds(base + k * ZROWS, ZROWS)], zsem).wait()

            @pl.when(s == NS - 1)
            def _():
                pltpu.make_async_copy(
                    zbuf.at[pl.ds(0, ROWS_TAIL)],
                    acc.at[pl.ds(ROWS_MAIN * NS, ROWS_TAIL)], zsem).wait()

            plsc.subcore_barrier()

            for r in range(NBUF):
                gather(r, r)

            @pl.loop(0, pl.cdiv(NJ, NBUF))
            def _(kk):
                for r in range(NBUF):
                    j = kk * NBUF + r

                    @pl.when(j < NJ)
                    def _():
                        gather_wait(j, r)
                        scatter(j, r)

                    @pl.when(j + NBUF < NJ)
                    def _():
                        scatter_wait(j, r)
                        gather(j + NBUF, r)

            for r in range(NBUF):
                scatter_wait(0, r)

            plsc.subcore_barrier()

            pltpu.async_copy(acc.at[pl.ds(base, ROWS_MAIN)],
                             msgs_hbm.at[pl.ds(base, ROWS_MAIN),
                                         pl.ds(t * D, D)], zsem)

            @pl.when(s == NS - 1)
            def _():
                pltpu.async_copy(
                    acc.at[pl.ds(ROWS_MAIN * NS, ROWS_TAIL)],
                    msgs_hbm.at[pl.ds(ROWS_MAIN * NS, ROWS_TAIL),
                                pl.ds(t * D, D)], zsem)

            if tt + 1 < TYPES_PER_SC:
                stage_idx(t + 1)

            pltpu.make_async_copy(acc.at[pl.ds(base, ROWS_MAIN)],
                                  msgs_hbm.at[pl.ds(base, ROWS_MAIN),
                                              pl.ds(t * D, D)], zsem).wait()

            @pl.when(s == NS - 1)
            def _():
                pltpu.make_async_copy(
                    acc.at[pl.ds(ROWS_MAIN * NS, ROWS_TAIL)],
                    msgs_hbm.at[pl.ds(ROWS_MAIN * NS, ROWS_TAIL),
                                pl.ds(t * D, D)], zsem).wait()


    return sc_kernel(node_values, edges_r, zeros)


BV = 5000


def _mm_body(msgs_ref, w_ref, b_ref, out_ref):
    out_ref[...] = b_ref[...] + jnp.dot(msgs_ref[...], w_ref[...],
                                        preferred_element_type=jnp.float32)


def _tc_matmul(msgs, W, b2):
    return pl.pallas_call(
        _mm_body,
        grid=(V // BV,),
        in_specs=[
            pl.BlockSpec((BV, T * D), lambda i: (i, 0)),
            pl.BlockSpec((T * D, D), lambda i: (0, 0)),
            pl.BlockSpec((1, D), lambda i: (0, 0)),
        ],
        out_specs=pl.BlockSpec((BV, D), lambda i: (i, 0)),
        out_shape=jax.ShapeDtypeStruct((V, D), jnp.float32),
        compiler_params=pltpu.CompilerParams(
            dimension_semantics=("parallel",)),
    )(msgs, W, b2)


def kernel(node_values, edges, W, b):
    edges_r = edges.astype(jnp.int32)
    zeros = jnp.zeros((ZROWS, D), jnp.float32)
    msgs = _sc_message_passing(node_values, edges_r, zeros)
    return _tc_matmul(msgs, W, b.reshape(1, D))
